# Initial kernel scaffold; baseline (speedup 1.0000x reference)
import jax
import jax.numpy as jnp
from jax.experimental import pallas as pl


def kernel(x, kernel, dst_local, dst_gidx, src_local, src_gidx):
    raise NotImplementedError("write your pallas kernel here")


# speedup vs baseline: 46.6856x; 46.6856x over previous
import jax
import jax.numpy as jnp
from jax import lax
from jax.experimental import pallas as pl
from jax.experimental.pallas import tpu as pltpu

_N_CLUSTERS = 50
_CLUSTER = 2000
_TOT = 3200
_T = 128
_D = 8
_K = 64


def _step(x_ref, t_ref, dl_ref, dg_ref, sl_ref, sg_ref, y_ref, bucket):
    c = pl.program_id(0)

    @pl.when(c == 0)
    def _():
        bucket[...] = jnp.zeros_like(bucket)

    dl = dl_ref[0, 0, :]
    dg = dg_ref[0, 0, :]
    sl = sl_ref[0, 0, :]
    sg = sg_ref[0, 0, :]

    tmat = t_ref[...]
    xb = x_ref[0]
    buck = bucket[...]

    ohg = (lax.broadcasted_iota(jnp.int32, (_K, _TOT), 1) == dg[:, None]
           ).astype(jnp.float32)
    inc = jnp.dot(ohg, buck, preferred_element_type=jnp.float32)
    conv_inc = jnp.dot(inc, tmat, preferred_element_type=jnp.float32)

    ybase = jnp.dot(xb, tmat, preferred_element_type=jnp.float32)

    ohd = (lax.broadcasted_iota(jnp.int32, (_CLUSTER, _K), 0) == dl[None, :]
           ).astype(jnp.float32)
    y = ybase + jnp.dot(ohd, conv_inc, preferred_element_type=jnp.float32)

    ohs = (lax.broadcasted_iota(jnp.int32, (_K, _CLUSTER), 1) == sl[:, None]
           ).astype(jnp.float32)
    out = jnp.dot(ohs, y, preferred_element_type=jnp.float32)

    ohb = (lax.broadcasted_iota(jnp.int32, (_TOT, _K), 0) == sg[None, :]
           ).astype(jnp.float32)
    bucket[...] = buck + jnp.dot(ohb, out, preferred_element_type=jnp.float32)

    y_ref[0] = y


def _toeplitz(fir):
    idx = jnp.arange(_T)
    diff = idx[None, :] - idx[:, None]
    mask = (diff >= 0) & (diff < _D)
    return jnp.where(mask, fir[jnp.clip(diff, 0, _D - 1)], 0.0).astype(jnp.float32)


def kernel(x, kernel, dst_local, dst_gidx, src_local, src_gidx):
    fir = kernel
    tmat = _toeplitz(fir)
    dl = dst_local.astype(jnp.int32).reshape(_N_CLUSTERS, 1, _K)
    dg = dst_gidx.astype(jnp.int32).reshape(_N_CLUSTERS, 1, _K)
    sl = src_local.astype(jnp.int32).reshape(_N_CLUSTERS, 1, _K)
    sg = src_gidx.astype(jnp.int32).reshape(_N_CLUSTERS, 1, _K)

    grid = (_N_CLUSTERS,)
    y = pl.pallas_call(
        _step,
        grid=grid,
        in_specs=[
            pl.BlockSpec((1, _CLUSTER, _T), lambda c: (0, c, 0)),
            pl.BlockSpec((_T, _T), lambda c: (0, 0)),
            pl.BlockSpec((1, 1, _K), lambda c: (c, 0, 0)),
            pl.BlockSpec((1, 1, _K), lambda c: (c, 0, 0)),
            pl.BlockSpec((1, 1, _K), lambda c: (c, 0, 0)),
            pl.BlockSpec((1, 1, _K), lambda c: (c, 0, 0)),
        ],
        out_specs=pl.BlockSpec((1, _CLUSTER, _T), lambda c: (0, c, 0)),
        out_shape=jax.ShapeDtypeStruct(x.shape, jnp.float32),
        scratch_shapes=[pltpu.VMEM((_TOT, _T), jnp.float32)],
        compiler_params=pltpu.CompilerParams(
            dimension_semantics=("arbitrary",),
        ),
    )(x, tmat, dl, dg, sl, sg)
    return y


# speedup vs baseline: 55.9713x; 1.1989x over previous
import jax
import jax.numpy as jnp
from jax import lax
from jax.experimental import pallas as pl
from jax.experimental.pallas import tpu as pltpu

_N_CLUSTERS = 50
_CLUSTER = 2000
_TOT = 3200
_T = 128
_D = 8
_K = 64


def _step(x_ref, t_ref, sgf_ref, dl_ref, dg_ref, sl_ref, y_ref, olog):
    c = pl.program_id(0)

    @pl.when(c == 0)
    def _():
        olog[...] = jnp.zeros_like(olog)

    dl = dl_ref[0, 0, :]
    dg = dg_ref[0, 0, :]
    sl = sl_ref[0, 0, :]
    sgf = sgf_ref[0, :]
    tmat = t_ref[...]
    xb = x_ref[0]

    eq = dg[:, None] == sgf[None, :]
    lt = lax.broadcasted_iota(jnp.int32, (_K, _TOT), 1) < c * _K
    m = (eq & lt).astype(jnp.bfloat16)
    inc = jnp.dot(m, olog[...], preferred_element_type=jnp.float32)

    ohd = (lax.broadcasted_iota(jnp.int32, (_CLUSTER, _K), 0) == dl[None, :]
           ).astype(jnp.bfloat16)
    z = xb + jnp.dot(ohd, inc, preferred_element_type=jnp.float32)
    y = jnp.dot(z, tmat, preferred_element_type=jnp.float32)

    ohs = (lax.broadcasted_iota(jnp.int32, (_K, _CLUSTER), 1) == sl[:, None]
           ).astype(jnp.bfloat16)
    out = jnp.dot(ohs, y, preferred_element_type=jnp.float32)
    olog[pl.ds(c * _K, _K), :] = out.astype(jnp.bfloat16)

    y_ref[0] = y


def _toeplitz(fir):
    idx = jnp.arange(_T)
    diff = idx[None, :] - idx[:, None]
    mask = (diff >= 0) & (diff < _D)
    return jnp.where(mask, fir[jnp.clip(diff, 0, _D - 1)], 0.0).astype(jnp.float32)


def kernel(x, kernel, dst_local, dst_gidx, src_local, src_gidx):
    fir = kernel
    tmat = _toeplitz(fir)
    dl = dst_local.astype(jnp.int32).reshape(_N_CLUSTERS, 1, _K)
    dg = dst_gidx.astype(jnp.int32).reshape(_N_CLUSTERS, 1, _K)
    sl = src_local.astype(jnp.int32).reshape(_N_CLUSTERS, 1, _K)
    sgf = src_gidx.astype(jnp.int32).reshape(1, _TOT)

    grid = (_N_CLUSTERS,)
    y = pl.pallas_call(
        _step,
        grid=grid,
        in_specs=[
            pl.BlockSpec((1, _CLUSTER, _T), lambda c: (0, c, 0)),
            pl.BlockSpec((_T, _T), lambda c: (0, 0)),
            pl.BlockSpec((1, _TOT), lambda c: (0, 0)),
            pl.BlockSpec((1, 1, _K), lambda c: (c, 0, 0)),
            pl.BlockSpec((1, 1, _K), lambda c: (c, 0, 0)),
            pl.BlockSpec((1, 1, _K), lambda c: (c, 0, 0)),
        ],
        out_specs=pl.BlockSpec((1, _CLUSTER, _T), lambda c: (0, c, 0)),
        out_shape=jax.ShapeDtypeStruct(x.shape, jnp.float32),
        scratch_shapes=[pltpu.VMEM((_TOT, _T), jnp.bfloat16)],
        compiler_params=pltpu.CompilerParams(
            dimension_semantics=("arbitrary",),
        ),
    )(x, tmat, sgf, dl, dg, sl)
    return y
